# baseline (speedup 1.0000x reference)
import functools

import jax
import jax.numpy as jnp
from jax import lax
from jax.experimental import pallas as pl
from jax.experimental.pallas import tpu as pltpu
from jax.experimental.pallas import tpu_sc as plsc

_VOCAB = 1000000
_DIM = 64
_UNITS = 256
_BATCH = 1024
_SEQ = 50

_NC = 2
_NS = 16
_NW = _NC * _NS
_CHUNK = 80
_NCHUNK = (_BATCH * _SEQ) // (_NW * _CHUNK)


def _sc_gather(table, idx3):
    mesh = plsc.VectorSubcoreMesh(core_axis_name="c", subcore_axis_name="s")

    @functools.partial(
        pl.kernel,
        mesh=mesh,
        compiler_params=pltpu.CompilerParams(use_tc_tiling_on_sc=False),
        out_type=jax.ShapeDtypeStruct((_NW, _NCHUNK, _CHUNK, _DIM), jnp.float32),
        scratch_types=[
            pltpu.VMEM((_NCHUNK, _CHUNK), jnp.int32),
            pltpu.VMEM((_NCHUNK, _CHUNK, _DIM), jnp.float32),
            pltpu.SemaphoreType.DMA,
        ],
    )
    def gather_kernel(table_hbm, idx_hbm, out_hbm, idx_v, rows_v, sem):
        wid = lax.axis_index("s") * _NC + lax.axis_index("c")
        pltpu.sync_copy(idx_hbm.at[wid], idx_v)
        copies = []
        for j in range(_NCHUNK):
            copies.append(
                pltpu.async_copy(table_hbm.at[idx_v.at[j]], rows_v.at[j], sem)
            )
        for c in copies:
            c.wait()
        pltpu.sync_copy(rows_v, out_hbm.at[wid])

    return gather_kernel(table, idx3)


def _gru_step(emb_ref, w_ref, u_ref, b_ref, h_ref):
    t = pl.program_id(0)

    @pl.when(t == 0)
    def _():
        h_ref[...] = jnp.zeros_like(h_ref)

    h = h_ref[...]
    xt = emb_ref[0]
    bb = b_ref[...]
    xp = jnp.dot(xt, w_ref[...], preferred_element_type=jnp.float32) + bb[0:1]
    rp = jnp.dot(h, u_ref[...], preferred_element_type=jnp.float32) + bb[1:2]
    xz = xp[:, :_UNITS]
    xr = xp[:, _UNITS:2 * _UNITS]
    xh = xp[:, 2 * _UNITS:]
    rz = rp[:, :_UNITS]
    rr = rp[:, _UNITS:2 * _UNITS]
    rh = rp[:, 2 * _UNITS:]
    z = jax.nn.sigmoid(xz + rz)
    r = jax.nn.sigmoid(xr + rr)
    hh = jnp.tanh(xh + r * rh)
    h_ref[...] = z * h + (1.0 - z) * hh


def _tc_gru(emb, W, U, b):
    return pl.pallas_call(
        _gru_step,
        grid=(_SEQ,),
        in_specs=[
            pl.BlockSpec((1, _BATCH, _DIM), lambda t: (t, 0, 0)),
            pl.BlockSpec((_DIM, 3 * _UNITS), lambda t: (0, 0)),
            pl.BlockSpec((_UNITS, 3 * _UNITS), lambda t: (0, 0)),
            pl.BlockSpec((2, 3 * _UNITS), lambda t: (0, 0)),
        ],
        out_specs=pl.BlockSpec((_BATCH, _UNITS), lambda t: (0, 0)),
        out_shape=jax.ShapeDtypeStruct((_BATCH, _UNITS), jnp.float32),
    )(emb, W, U, b)


def kernel(x, emb_table, W, U, b):
    idx = jnp.transpose(x).reshape(_NW, _NCHUNK, _CHUNK)
    rows = _sc_gather(emb_table, idx)
    emb = rows.reshape(_SEQ, _BATCH, _DIM)
    return _tc_gru(emb, W, U, b)
